# baseline (speedup 1.0000x reference)
import functools

import jax
import jax.numpy as jnp
from jax import lax
from jax.experimental import pallas as pl
from jax.experimental.pallas import tpu as pltpu
from jax.experimental.pallas import tpu_sc as plsc

D = 128
NC = 2
NS = 16
NW = NC * NS
BLK = 2048
NB = 16

H1 = 16384
H2 = 17920
GC1 = 128
GC2 = 112


def _make_gather(rows: int, chunk: int):
    assert rows % (NW * chunk) == 0 and chunk % 8 == 0 and chunk <= 128
    per_w = rows // NW
    n = per_w // chunk
    mesh = plsc.VectorSubcoreMesh(core_axis_name="c", subcore_axis_name="s")

    @functools.partial(
        pl.kernel,
        mesh=mesh,
        out_type=jax.ShapeDtypeStruct((rows, D), jnp.float32),
        scratch_types=[
            pltpu.VMEM((per_w,), jnp.int32),
            [pltpu.VMEM((chunk, D), jnp.float32) for _ in range(n)],
            [pltpu.SemaphoreType.DMA for _ in range(n)],
            [pltpu.SemaphoreType.DMA for _ in range(n)],
        ],
    )
    def gather_kernel(table_hbm, ids_hbm, out_hbm, idx_v, bufs, gsem, wsem):
        wid = lax.axis_index("s") * NC + lax.axis_index("c")
        base = wid * per_w
        pltpu.sync_copy(ids_hbm.at[pl.ds(pl.multiple_of(base, 8), per_w)],
                        idx_v)
        gd = [pltpu.async_copy(
                  table_hbm.at[idx_v.at[pl.ds(k * chunk, chunk)]],
                  bufs[k], gsem[k]) for k in range(n)]
        wbd = []
        for k in range(n):
            gd[k].wait()
            off = pl.multiple_of(base + k * chunk, 8)
            wbd.append(pltpu.async_copy(
                bufs[k], out_hbm.at[pl.ds(off, chunk), :], wsem[k]))
        for k in range(n):
            wbd[k].wait()

    return gather_kernel


def _tc_body(g1_ref, g2_ref, ht_ref, htt_ref, eat_ref, wd_ref, we_ref,
             bias_ref, lo_ref, hi_ref, lo_col_ref, hi_col_ref, out_ref,
             s_scr, den_scr, acc_scr):
    f32 = jnp.float32
    i = pl.program_id(0)
    B = lo_ref.shape[1]
    j = lax.rem(i, NB)

    emb = jnp.where(j < NB // 2, g1_ref[...], g2_ref[...])
    pos = lax.broadcasted_iota(jnp.int32, (BLK, B), 0) + j * BLK
    onehot = jnp.logical_and(pos >= lo_ref[...], pos < hi_ref[...]).astype(f32)

    @pl.when(i < NB)
    def _():
        s = jnp.exp(jnp.dot(emb, eat_ref[...],
                            preferred_element_type=f32))
        s_scr[pl.ds(j * BLK, BLK), :] = s
        part = lax.dot_general(onehot, s, (((0,), (0,)), ((), ())),
                               preferred_element_type=f32)
        den_scr[...] = jnp.where(i == 0, part, den_scr[...] + part)

    @pl.when(i >= NB)
    def _():
        s = s_scr[pl.ds(j * BLK, BLK), :]
        att = s / jnp.dot(onehot, den_scr[...],
                          preferred_element_type=f32)
        hd = jnp.dot(emb, htt_ref[...],
                     preferred_element_type=f32)
        oh2 = jnp.concatenate([onehot, onehot], axis=1)
        r2 = lax.broadcasted_iota(jnp.int32, (2 * B, 2), 0)
        c2 = lax.broadcasted_iota(jnp.int32, (2 * B, 2), 1)
        sel = ((r2 < B) == (c2 == 0)).astype(f32)
        nums = jnp.dot(hd * oh2, sel,
                       preferred_element_type=f32)
        ht = ht_ref[...]
        nh2 = jnp.sum(ht * ht, axis=1, keepdims=True)
        nrm_h = jnp.dot(onehot, nh2[0:B],
                        preferred_element_type=f32)
        nrm_t = jnp.dot(onehot, nh2[B:2 * B],
                        preferred_element_type=f32)
        en = jnp.sqrt(jnp.dot(emb * emb, jnp.ones((D, 1), f32),
                              preferred_element_type=f32))
        sim_h = nums[:, 0:1] / (en * jnp.sqrt(nrm_h) + 1e-8)
        sim_t = nums[:, 1:2] / (en * jnp.sqrt(nrm_t) + 1e-8)
        dist = (1.0 - 0.5 * (sim_h + sim_t)) * 0.5
        x1 = jnp.dot(emb, wd_ref[...], preferred_element_type=f32)
        x2 = jnp.dot(emb, we_ref[...], preferred_element_type=f32)
        c = jnp.maximum(dist * x1 + att * x2 + bias_ref[...], 0.0)
        part = lax.dot_general(onehot, c, (((0,), (0,)), ((), ())),
                               preferred_element_type=f32)
        acc_scr[...] = jnp.where(i == NB, part, acc_scr[...] + part)

    @pl.when(i == 2 * NB - 1)
    def _():
        counts = (hi_col_ref[...] - lo_col_ref[...]).astype(f32)
        out_ref[...] = (acc_scr[...] * 0.001 / jnp.maximum(counts, 1.0)
                        + ht_ref[B:2 * B, :] - ht_ref[0:B, :])


def kernel(table, w_d_w, w_d_b, w_e_w, w_e_b, eat_w, eat_b,
           flat_ids, cu_seqlens, head_ids, tail_ids):
    T = flat_ids.shape[0]
    B = head_ids.shape[0]
    f32 = jnp.float32
    assert T == NB * BLK and H1 == T // 2 and H1 + H2 >= T + 2 * B

    ids_all = jnp.concatenate([
        flat_ids.astype(jnp.int32),
        head_ids.astype(jnp.int32),
        tail_ids.astype(jnp.int32),
        jnp.arange(H1 + H2 - T - 2 * B, dtype=jnp.int32),
    ])

    g1 = _make_gather(H1, GC1)(table, ids_all[:H1])
    g2 = _make_gather(H2, GC2)(table, ids_all[H1:])

    cu = cu_seqlens.astype(jnp.int32)
    lo = cu[:B].reshape(1, B)
    hi = cu[1:B + 1].reshape(1, B)
    lo_col = cu[:B].reshape(B, 1)
    hi_col = cu[1:B + 1].reshape(B, 1)
    bias = (w_d_b + w_e_b).reshape(1, D)
    ht_rows = lax.slice(g2, (T - H1, 0), (T - H1 + 2 * B, D))
    htt = ht_rows.T

    half = NB // 2
    full = lambda shape: pl.BlockSpec(shape, lambda i: (0, 0))
    ht_block = (T - H1) // (2 * B)

    out = pl.pallas_call(
        functools.partial(_tc_body),
        grid=(2 * NB,),
        in_specs=[
            pl.BlockSpec((BLK, D),
                         lambda i: (jnp.minimum(lax.rem(i, NB), half - 1), 0)),
            pl.BlockSpec((BLK, D),
                         lambda i: (jnp.clip(lax.rem(i, NB) - half, 0,
                                             half - 1), 0)),
            pl.BlockSpec((2 * B, D), lambda i: (ht_block, 0)),
            full((D, 2 * B)),
            full((D, 1)),
            full((D, D)), full((D, D)),
            full((1, D)),
            full((1, B)), full((1, B)),
            full((B, 1)), full((B, 1)),
        ],
        out_specs=pl.BlockSpec((B, D), lambda i: (0, 0)),
        out_shape=jax.ShapeDtypeStruct((B, D), f32),
        scratch_shapes=[
            pltpu.VMEM((T, 1), f32),
            pltpu.VMEM((B, 1), f32),
            pltpu.VMEM((B, D), f32),
        ],
    )(g1, g2, g2, htt, eat_w, w_d_w, w_e_w, bias, lo, hi, lo_col, hi_col)
    return out


# speedup vs baseline: 1.0766x; 1.0766x over previous
import functools

import jax
import jax.numpy as jnp
from jax import lax
from jax.experimental import pallas as pl
from jax.experimental.pallas import tpu as pltpu
from jax.experimental.pallas import tpu_sc as plsc

D = 128
NC = 2
NS = 16
NW = NC * NS
BLK = 2048

H1 = 16384
H2 = 17920
GC1 = 128
GC2 = 112


def _make_gather(rows: int, chunk: int):
    assert rows % (NW * chunk) == 0 and chunk % 8 == 0 and chunk <= 128
    per_w = rows // NW
    n = per_w // chunk
    mesh = plsc.VectorSubcoreMesh(core_axis_name="c", subcore_axis_name="s")

    @functools.partial(
        pl.kernel,
        mesh=mesh,
        out_type=jax.ShapeDtypeStruct((rows, D), jnp.float32),
        scratch_types=[
            pltpu.VMEM((per_w,), jnp.int32),
            [pltpu.VMEM((chunk, D), jnp.float32) for _ in range(n)],
            [pltpu.SemaphoreType.DMA for _ in range(n)],
            [pltpu.SemaphoreType.DMA for _ in range(n)],
        ],
    )
    def gather_kernel(table_hbm, ids_hbm, out_hbm, idx_v, bufs, gsem, wsem):
        wid = lax.axis_index("s") * NC + lax.axis_index("c")
        base = wid * per_w
        pltpu.sync_copy(ids_hbm.at[pl.ds(pl.multiple_of(base, 8), per_w)],
                        idx_v)
        gd = [pltpu.async_copy(
                  table_hbm.at[idx_v.at[pl.ds(k * chunk, chunk)]],
                  bufs[k], gsem[k]) for k in range(n)]
        wbd = []
        for k in range(n):
            gd[k].wait()
            off = pl.multiple_of(base + k * chunk, 8)
            wbd.append(pltpu.async_copy(
                bufs[k], out_hbm.at[pl.ds(off, chunk), :], wsem[k]))
        for k in range(n):
            wbd[k].wait()

    return gather_kernel


def _pass_a_body(tok0, g_ref, eat_ref, lo_ref, hi_ref, s_ref, den_ref):
    i = pl.program_id(0)
    f32 = jnp.float32
    B = lo_ref.shape[1]
    emb = g_ref[...]
    s = jnp.exp(jnp.dot(emb, eat_ref[...], preferred_element_type=f32))
    s_ref[...] = s
    pos = lax.broadcasted_iota(jnp.int32, (BLK, B), 0) + tok0 + i * BLK
    onehot = jnp.logical_and(pos >= lo_ref[...], pos < hi_ref[...]).astype(f32)
    part = lax.dot_general(onehot, s, (((0,), (0,)), ((), ())),
                           preferred_element_type=f32)

    @pl.when(i == 0)
    def _():
        den_ref[...] = part

    @pl.when(i > 0)
    def _():
        den_ref[...] += part


def _pass_b_body(tok0, final, g_ref, ht_ref, htt_ref, nhnt_ref, sel_ref,
                 s_ref, den1_ref, den2_ref, lo_ref, hi_ref, lo_col_ref,
                 hi_col_ref, wd_ref, we_ref, bias_ref, accin_ref, out_ref,
                 acc_scr):
    f32 = jnp.float32
    i = pl.program_id(0)
    nblk = pl.num_programs(0)
    B = lo_ref.shape[1]

    emb = g_ref[...]
    s = s_ref[...]
    pos = lax.broadcasted_iota(jnp.int32, (BLK, B), 0) + tok0 + i * BLK
    onehot = jnp.logical_and(pos >= lo_ref[...], pos < hi_ref[...]).astype(f32)
    den_col = den1_ref[...] + den2_ref[...]
    att = s / jnp.dot(onehot, den_col, preferred_element_type=f32)
    hd = jnp.dot(emb, htt_ref[...], preferred_element_type=f32)
    oh2 = jnp.concatenate([onehot, onehot], axis=1)
    nums = jnp.dot(hd * oh2, sel_ref[...],
                   preferred_element_type=f32)
    nrm2 = jnp.dot(onehot, nhnt_ref[...],
                   preferred_element_type=f32)
    en = jnp.sqrt(jnp.dot(emb * emb, jnp.ones((D, 1), f32),
                          preferred_element_type=f32))
    sims = nums / (en * jnp.sqrt(nrm2) + 1e-8)
    dist = (1.0 - 0.5 * (sims[:, 0:1] + sims[:, 1:2])) * 0.5
    x1 = jnp.dot(emb, wd_ref[...], preferred_element_type=f32)
    x2 = jnp.dot(emb, we_ref[...], preferred_element_type=f32)
    c = jnp.maximum(dist * x1 + att * x2 + bias_ref[...], 0.0)
    part = lax.dot_general(onehot, c, (((0,), (0,)), ((), ())),
                           preferred_element_type=f32)

    @pl.when(i == 0)
    def _():
        acc_scr[...] = part + accin_ref[...]

    @pl.when(i > 0)
    def _():
        acc_scr[...] += part

    @pl.when(i == nblk - 1)
    def _():
        if final:
            counts = (hi_col_ref[...] - lo_col_ref[...]).astype(f32)
            out_ref[...] = (acc_scr[...] * 0.001 / jnp.maximum(counts, 1.0)
                            + ht_ref[B:2 * B, :] - ht_ref[0:B, :])
        else:
            out_ref[...] = acc_scr[...]


def kernel(table, w_d_w, w_d_b, w_e_w, w_e_b, eat_w, eat_b,
           flat_ids, cu_seqlens, head_ids, tail_ids):
    T = flat_ids.shape[0]
    B = head_ids.shape[0]
    f32 = jnp.float32
    assert H1 % BLK == 0 and (T - H1) % BLK == 0 and H1 + H2 >= T + 2 * B

    ids_all = jnp.concatenate([
        flat_ids.astype(jnp.int32),
        head_ids.astype(jnp.int32),
        tail_ids.astype(jnp.int32),
        jnp.arange(H1 + H2 - T - 2 * B, dtype=jnp.int32),
    ])

    g1 = _make_gather(H1, GC1)(table, ids_all[:H1])
    g2 = _make_gather(H2, GC2)(table, ids_all[H1:])

    cu = cu_seqlens.astype(jnp.int32)
    lo = cu[:B].reshape(1, B)
    hi = cu[1:B + 1].reshape(1, B)
    lo_col = cu[:B].reshape(B, 1)
    hi_col = cu[1:B + 1].reshape(B, 1)
    bias = (w_d_b + w_e_b).reshape(1, D)

    n1 = H1 // BLK
    n2 = (T - H1) // BLK
    row_spec = pl.BlockSpec((1, B), lambda i: (0, 0))
    full = lambda shape: pl.BlockSpec(shape, lambda i: (0, 0))

    def pass_a(g, nblk, tok0):
        return pl.pallas_call(
            functools.partial(_pass_a_body, tok0),
            grid=(nblk,),
            in_specs=[
                pl.BlockSpec((BLK, D), lambda i: (i, 0)),
                full((D, 1)), row_spec, row_spec,
            ],
            out_specs=[pl.BlockSpec((BLK, 1), lambda i: (i, 0)),
                       full((B, 1))],
            out_shape=[jax.ShapeDtypeStruct((nblk * BLK, 1), f32),
                       jax.ShapeDtypeStruct((B, 1), f32)],
        )(g, eat_w, lo, hi)

    s1, den1_col = pass_a(g1, n1, 0)
    s2, den2_col = pass_a(g2, n2, H1)

    ht_rows = lax.slice(g2, (T - H1, 0), (T - H1 + 2 * B, D))
    htt = ht_rows.T
    nh_nt = jnp.sum(ht_rows * ht_rows, axis=1).reshape(2, B).T
    sel = jnp.concatenate([
        jnp.concatenate([jnp.ones((B, 1), f32), jnp.zeros((B, 1), f32)], 1),
        jnp.concatenate([jnp.zeros((B, 1), f32), jnp.ones((B, 1), f32)], 1),
    ], 0)

    ht_spec = pl.BlockSpec((2 * B, D), lambda i: ((T - H1) // (2 * B), 0))

    def pass_b(g, s, nblk, tok0, accin, final):
        return pl.pallas_call(
            functools.partial(_pass_b_body, tok0, final),
            grid=(nblk,),
            in_specs=[
                pl.BlockSpec((BLK, D), lambda i: (i, 0)),
                ht_spec,
                full((D, 2 * B)), full((B, 2)),
                full((2 * B, 2)),
                pl.BlockSpec((BLK, 1), lambda i: (i, 0)),
                full((B, 1)), full((B, 1)),
                row_spec, row_spec,
                full((B, 1)), full((B, 1)),
                full((D, D)), full((D, D)),
                full((1, D)),
                full((B, D)),
            ],
            out_specs=pl.BlockSpec((B, D), lambda i: (0, 0)),
            out_shape=jax.ShapeDtypeStruct((B, D), f32),
            scratch_shapes=[pltpu.VMEM((B, D), f32)],
        )(g, g2, htt, nh_nt, sel, s, den1_col, den2_col, lo, hi, lo_col,
          hi_col, w_d_w, w_e_w, bias, accin)

    acc1 = pass_b(g1, s1, n1, 0, jnp.zeros((B, D), f32), False)
    out = pass_b(g2, s2, n2, H1, acc1, True)
    return out


# speedup vs baseline: 1.0951x; 1.0172x over previous
import functools

import jax
import jax.numpy as jnp
from jax import lax
from jax.experimental import pallas as pl
from jax.experimental.pallas import tpu as pltpu
from jax.experimental.pallas import tpu_sc as plsc

D = 128
NC = 2
NS = 16
NW = NC * NS
BLK = 2048

H1 = 16384
H2 = 17920
GC1 = 128
GC2 = 112


def _make_gather(rows: int, chunk: int):
    assert rows % (NW * chunk) == 0 and chunk % 8 == 0 and chunk <= 128
    per_w = rows // NW
    n = per_w // chunk
    mesh = plsc.VectorSubcoreMesh(core_axis_name="c", subcore_axis_name="s")

    @functools.partial(
        pl.kernel,
        mesh=mesh,
        out_type=jax.ShapeDtypeStruct((rows, D), jnp.float32),
        scratch_types=[
            pltpu.VMEM((per_w,), jnp.int32),
            [pltpu.VMEM((chunk, D), jnp.float32) for _ in range(n)],
            [pltpu.SemaphoreType.DMA for _ in range(n)],
            [pltpu.SemaphoreType.DMA for _ in range(n)],
        ],
    )
    def gather_kernel(table_hbm, ids_hbm, out_hbm, idx_v, bufs, gsem, wsem):
        wid = lax.axis_index("s") * NC + lax.axis_index("c")
        base = wid * per_w
        pltpu.sync_copy(ids_hbm.at[pl.ds(pl.multiple_of(base, 8), per_w)],
                        idx_v)
        gd = [pltpu.async_copy(
                  table_hbm.at[idx_v.at[pl.ds(k * chunk, chunk)]],
                  bufs[k], gsem[k]) for k in range(n)]
        wbd = []
        for k in range(n):
            gd[k].wait()
            off = pl.multiple_of(base + k * chunk, 8)
            wbd.append(pltpu.async_copy(
                bufs[k], out_hbm.at[pl.ds(off, chunk), :], wsem[k]))
        for k in range(n):
            wbd[k].wait()

    return gather_kernel


def _pass_a_body(tok0, g_ref, eat_ref, lo_ref, hi_ref, s_ref, den_ref):
    i = pl.program_id(0)
    f32 = jnp.float32
    B = lo_ref.shape[1]
    emb = g_ref[...]
    s = jnp.exp(jnp.dot(emb, eat_ref[...], preferred_element_type=f32))
    s_ref[...] = s
    pos = lax.broadcasted_iota(jnp.int32, (BLK, B), 0) + tok0 + i * BLK
    onehot = jnp.logical_and(pos >= lo_ref[...], pos < hi_ref[...]).astype(f32)
    part = lax.dot_general(onehot, s, (((0,), (0,)), ((), ())),
                           preferred_element_type=f32)

    @pl.when(i == 0)
    def _():
        den_ref[...] = part

    @pl.when(i > 0)
    def _():
        den_ref[...] += part


def _pass_b_body(tok0, final, g_ref, ht_ref, htt_ref, nhnt_ref, sel_ref,
                 s_ref, den1_ref, den2_ref, lo_ref, hi_ref, lo_col_ref,
                 hi_col_ref, wcat_ref, bias_ref, accin_ref, out_ref,
                 acc_scr):
    f32 = jnp.float32
    i = pl.program_id(0)
    nblk = pl.num_programs(0)
    B = lo_ref.shape[1]

    emb = g_ref[...]
    s = s_ref[...]
    pos = lax.broadcasted_iota(jnp.int32, (BLK, B), 0) + tok0 + i * BLK
    onehot = jnp.logical_and(pos >= lo_ref[...], pos < hi_ref[...]).astype(f32)
    den_col = den1_ref[...] + den2_ref[...]
    att = s / jnp.dot(onehot, den_col, preferred_element_type=f32)
    hd = jnp.dot(emb, htt_ref[...], preferred_element_type=f32)
    oh2 = jnp.concatenate([onehot, onehot], axis=1)
    nums = jnp.dot(hd * oh2, sel_ref[...],
                   preferred_element_type=f32)
    nrm2 = jnp.dot(onehot, nhnt_ref[...],
                   preferred_element_type=f32)
    en = jnp.sqrt(jnp.dot(emb * emb, jnp.ones((D, 1), f32),
                          preferred_element_type=f32))
    sims = nums / (en * jnp.sqrt(nrm2) + 1e-8)
    dist = (1.0 - 0.5 * (sims[:, 0:1] + sims[:, 1:2])) * 0.5
    x = jnp.dot(emb, wcat_ref[...], preferred_element_type=f32)
    c = jnp.maximum(dist * x[:, :D] + att * x[:, D:] + bias_ref[...], 0.0)
    part = lax.dot_general(onehot, c, (((0,), (0,)), ((), ())),
                           preferred_element_type=f32)

    @pl.when(i == 0)
    def _():
        acc_scr[...] = part + accin_ref[...]

    @pl.when(i > 0)
    def _():
        acc_scr[...] += part

    @pl.when(i == nblk - 1)
    def _():
        if final:
            counts = (hi_col_ref[...] - lo_col_ref[...]).astype(f32)
            out_ref[...] = (acc_scr[...] * 0.001 / jnp.maximum(counts, 1.0)
                            + ht_ref[B:2 * B, :] - ht_ref[0:B, :])
        else:
            out_ref[...] = acc_scr[...]


def kernel(table, w_d_w, w_d_b, w_e_w, w_e_b, eat_w, eat_b,
           flat_ids, cu_seqlens, head_ids, tail_ids):
    T = flat_ids.shape[0]
    B = head_ids.shape[0]
    f32 = jnp.float32
    assert H1 % BLK == 0 and (T - H1) % BLK == 0 and H1 + H2 >= T + 2 * B

    ids_all = jnp.concatenate([
        flat_ids.astype(jnp.int32),
        head_ids.astype(jnp.int32),
        tail_ids.astype(jnp.int32),
        jnp.arange(H1 + H2 - T - 2 * B, dtype=jnp.int32),
    ])

    g1 = _make_gather(H1, GC1)(table, ids_all[:H1])
    g2 = _make_gather(H2, GC2)(table, ids_all[H1:])

    cu = cu_seqlens.astype(jnp.int32)
    lo = cu[:B].reshape(1, B)
    hi = cu[1:B + 1].reshape(1, B)
    lo_col = cu[:B].reshape(B, 1)
    hi_col = cu[1:B + 1].reshape(B, 1)
    bias = (w_d_b + w_e_b).reshape(1, D)
    wcat = jnp.concatenate([w_d_w, w_e_w], axis=1)

    n1 = H1 // BLK
    n2 = (T - H1) // BLK
    row_spec = pl.BlockSpec((1, B), lambda i: (0, 0))
    full = lambda shape: pl.BlockSpec(shape, lambda i: (0, 0))

    def pass_a(g, nblk, tok0):
        return pl.pallas_call(
            functools.partial(_pass_a_body, tok0),
            grid=(nblk,),
            in_specs=[
                pl.BlockSpec((BLK, D), lambda i: (i, 0)),
                full((D, 1)), row_spec, row_spec,
            ],
            out_specs=[pl.BlockSpec((BLK, 1), lambda i: (i, 0)),
                       full((B, 1))],
            out_shape=[jax.ShapeDtypeStruct((nblk * BLK, 1), f32),
                       jax.ShapeDtypeStruct((B, 1), f32)],
        )(g, eat_w, lo, hi)

    s1, den1_col = pass_a(g1, n1, 0)
    s2, den2_col = pass_a(g2, n2, H1)

    ht_rows = lax.slice(g2, (T - H1, 0), (T - H1 + 2 * B, D))
    htt = ht_rows.T
    nh_nt = jnp.sum(ht_rows * ht_rows, axis=1).reshape(2, B).T
    sel = jnp.concatenate([
        jnp.concatenate([jnp.ones((B, 1), f32), jnp.zeros((B, 1), f32)], 1),
        jnp.concatenate([jnp.zeros((B, 1), f32), jnp.ones((B, 1), f32)], 1),
    ], 0)

    ht_spec = pl.BlockSpec((2 * B, D), lambda i: ((T - H1) // (2 * B), 0))

    def pass_b(g, s, nblk, tok0, accin, final):
        return pl.pallas_call(
            functools.partial(_pass_b_body, tok0, final),
            grid=(nblk,),
            in_specs=[
                pl.BlockSpec((BLK, D), lambda i: (i, 0)),
                ht_spec,
                full((D, 2 * B)), full((B, 2)),
                full((2 * B, 2)),
                pl.BlockSpec((BLK, 1), lambda i: (i, 0)),
                full((B, 1)), full((B, 1)),
                row_spec, row_spec,
                full((B, 1)), full((B, 1)),
                full((D, 2 * D)),
                full((1, D)),
                full((B, D)),
            ],
            out_specs=pl.BlockSpec((B, D), lambda i: (0, 0)),
            out_shape=jax.ShapeDtypeStruct((B, D), f32),
            scratch_shapes=[pltpu.VMEM((B, D), f32)],
        )(g, g2, htt, nh_nt, sel, s, den1_col, den2_col, lo, hi, lo_col,
          hi_col, wcat, bias, accin)

    acc1 = pass_b(g1, s1, n1, 0, jnp.zeros((B, D), f32), False)
    out = pass_b(g2, s2, n2, H1, acc1, True)
    return out
